# baseline (speedup 1.0000x reference)
import functools

import jax
import jax.numpy as jnp
from jax import lax
from jax.experimental import pallas as pl
from jax.experimental.pallas import tpu as pltpu
from jax.experimental.pallas import tpu_sc as plsc

B = 4096
T = 200
D = 32
N_LOOKUPS = B * T
N_ELEMS = N_LOOKUPS * D

_NW = 32
_PER_W = N_LOOKUPS // _NW
_IDX_ROWS = _PER_W // 128
_CHUNK = 1024
_NCH = _PER_W // _CHUNK
_GPC = _CHUNK // 128


@functools.lru_cache(maxsize=1)
def _sc_gather_build():
    mesh = plsc.VectorSubcoreMesh(core_axis_name="c", subcore_axis_name="s")

    @functools.partial(
        pl.kernel,
        mesh=mesh,
        compiler_params=pltpu.CompilerParams(use_tc_tiling_on_sc=False),
        out_type=jax.ShapeDtypeStruct((N_LOOKUPS, D), jnp.float32),
        scratch_types=[
            pltpu.VMEM((_IDX_ROWS, 128), jnp.int32),
            pltpu.VMEM((_CHUNK, D), jnp.float32),
            pltpu.SemaphoreType.DMA,
        ],
    )
    def k(table_hbm, ids_hbm, out_hbm, idx_v, rows_v, sem):
        wid = lax.axis_index("s") * 2 + lax.axis_index("c")
        pltpu.sync_copy(ids_hbm.at[pl.ds(wid * _IDX_ROWS, _IDX_ROWS)], idx_v)

        def chunk_body(cc, carry):
            handles = []
            for j in range(_GPC):
                handles.append(pltpu.async_copy(
                    table_hbm.at[idx_v.at[cc * _GPC + j]],
                    rows_v.at[pl.ds(j * 128, 128)],
                    sem,
                ))
            for h in handles:
                h.wait()
            pltpu.sync_copy(
                rows_v,
                out_hbm.at[pl.ds(wid * _PER_W + cc * _CHUNK, _CHUNK)],
            )
            return carry

        lax.fori_loop(0, _NCH, chunk_body, 0)

    return k


_NROWS = N_ELEMS // 128
_RB = 512
_RBA = 1024

_KS0 = 0
_KS1 = 1234
_KS2 = _KS0 ^ _KS1 ^ 0x1BD11BDA
_ROT = ((13, 15, 26, 6), (17, 29, 16, 24))
_MAG = 5.0 / 80.0


def _threefry_bits(x1):
    ks = (jnp.uint32(_KS0), jnp.uint32(_KS1), jnp.uint32(_KS2))
    x0 = jnp.zeros_like(x1) + ks[0]
    x1 = x1 + ks[1]
    for i in range(5):
        for r in _ROT[i % 2]:
            x0 = x0 + x1
            x1 = (x1 << jnp.uint32(r)) | (x1 >> jnp.uint32(32 - r))
            x1 = x0 ^ x1
        x0 = x0 + ks[(i + 1) % 3]
        x1 = x1 + ks[(i + 2) % 3] + jnp.uint32(i + 1)
    return x0 ^ x1


def _noise_gen_body(out_ref):
    i = pl.program_id(0)
    base = i * (_RB * 128)
    row = lax.broadcasted_iota(jnp.int32, (_RB, 128), 0)
    col = lax.broadcasted_iota(jnp.int32, (_RB, 128), 1)
    f = (base + row * 128 + col).astype(jnp.uint32)
    bits = _threefry_bits(f)
    fb = (bits >> jnp.uint32(9)) | jnp.uint32(0x3F800000)
    u = lax.bitcast_convert_type(fb, jnp.float32) - jnp.float32(1.0)
    u = jnp.maximum(jnp.float32(-1.0),
                    u * jnp.float32(2.0) - jnp.float32(1.0))
    out_ref[...] = u * jnp.float32(_MAG)


_noise_gen = pl.pallas_call(
    _noise_gen_body,
    grid=(_NROWS // _RB,),
    out_specs=pl.BlockSpec((_RB, 128), lambda i: (i, 0)),
    out_shape=jax.ShapeDtypeStruct((_NROWS, 128), jnp.float32),
)


def _add_body(emb_ref, noise_ref, out_ref):
    out_ref[...] = emb_ref[...] + noise_ref[...]


_add = pl.pallas_call(
    _add_body,
    grid=(_NROWS // _RBA,),
    in_specs=[pl.BlockSpec((_RBA, 128), lambda i: (i, 0)),
              pl.BlockSpec((_RBA, 128), lambda i: (i, 0))],
    out_specs=pl.BlockSpec((_RBA, 128), lambda i: (i, 0)),
    out_shape=jax.ShapeDtypeStruct((_NROWS, 128), jnp.float32),
)


def kernel(input_ids, table):
    ids2d = input_ids.reshape(N_LOOKUPS // 128, 128)
    noise = _noise_gen()
    embeds = _sc_gather_build()(table, ids2d)
    out = _add(embeds.reshape(_NROWS, 128), noise)
    return out.reshape(B, T, D)


# speedup vs baseline: 1.0686x; 1.0686x over previous
import functools

import jax
import jax.numpy as jnp
from jax import lax
from jax.experimental import pallas as pl
from jax.experimental.pallas import tpu as pltpu
from jax.experimental.pallas import tpu_sc as plsc

B = 4096
T = 200
D = 32
N_LOOKUPS = B * T
N_ELEMS = N_LOOKUPS * D

_NW = 32
_PER_W = N_LOOKUPS // _NW
_IDX_ROWS = _PER_W // 128
_CHUNK = 1024
_NCH = _PER_W // _CHUNK
_GPC = _CHUNK // 128


@functools.lru_cache(maxsize=1)
def _sc_gather_build():
    mesh = plsc.VectorSubcoreMesh(core_axis_name="c", subcore_axis_name="s")

    @functools.partial(
        pl.kernel,
        mesh=mesh,
        compiler_params=pltpu.CompilerParams(use_tc_tiling_on_sc=False),
        out_type=jax.ShapeDtypeStruct((N_LOOKUPS, D), jnp.float32),
        scratch_types=[
            pltpu.VMEM((_IDX_ROWS, 128), jnp.int32),
            pltpu.VMEM((_CHUNK, D), jnp.float32),
            pltpu.SemaphoreType.DMA,
        ],
    )
    def k(table_hbm, ids_hbm, out_hbm, idx_v, rows_v, sem):
        wid = lax.axis_index("s") * 2 + lax.axis_index("c")
        pltpu.sync_copy(ids_hbm.at[pl.ds(wid * _IDX_ROWS, _IDX_ROWS)], idx_v)

        def chunk_body(cc, carry):
            handles = []
            for j in range(_GPC):
                handles.append(pltpu.async_copy(
                    table_hbm.at[idx_v.at[cc * _GPC + j]],
                    rows_v.at[pl.ds(j * 128, 128)],
                    sem,
                ))
            for h in handles:
                h.wait()
            pltpu.sync_copy(
                rows_v,
                out_hbm.at[pl.ds(wid * _PER_W + cc * _CHUNK, _CHUNK)],
            )
            return carry

        lax.fori_loop(0, _NCH, chunk_body, 0)

    return k


_TB = 4

_KS0 = 0
_KS1 = 1234
_KS2 = _KS0 ^ _KS1 ^ 0x1BD11BDA
_ROT = ((13, 15, 26, 6), (17, 29, 16, 24))
_MAG = 5.0 / 80.0


def _threefry_bits(x1):
    ks = (jnp.uint32(_KS0), jnp.uint32(_KS1), jnp.uint32(_KS2))
    x0 = jnp.zeros_like(x1) + ks[0]
    x1 = x1 + ks[1]
    for i in range(5):
        for r in _ROT[i % 2]:
            x0 = x0 + x1
            x1 = (x1 << jnp.uint32(r)) | (x1 >> jnp.uint32(32 - r))
            x1 = x0 ^ x1
        x0 = x0 + ks[(i + 1) % 3]
        x1 = x1 + ks[(i + 2) % 3] + jnp.uint32(i + 1)
    return x0 ^ x1


def _noise_add_body(emb_ref, out_ref):
    i = pl.program_id(0)
    t = lax.broadcasted_iota(jnp.int32, (_TB, D, B), 0) + i * _TB
    d = lax.broadcasted_iota(jnp.int32, (_TB, D, B), 1)
    b = lax.broadcasted_iota(jnp.int32, (_TB, D, B), 2)
    f = (b * (T * D) + t * D + d).astype(jnp.uint32)
    bits = _threefry_bits(f)
    fb = (bits >> jnp.uint32(9)) | jnp.uint32(0x3F800000)
    u = lax.bitcast_convert_type(fb, jnp.float32) - jnp.float32(1.0)
    u = jnp.maximum(jnp.float32(-1.0),
                    u * jnp.float32(2.0) - jnp.float32(1.0))
    out_ref[...] = emb_ref[...] + u * jnp.float32(_MAG)


_noise_add = pl.pallas_call(
    _noise_add_body,
    grid=(T // _TB,),
    in_specs=[pl.BlockSpec((_TB, D, B), lambda i: (i, 0, 0))],
    out_specs=pl.BlockSpec((_TB, D, B), lambda i: (i, 0, 0)),
    out_shape=jax.ShapeDtypeStruct((T, D, B), jnp.float32),
)


def kernel(input_ids, table):
    ids2d = input_ids.reshape(N_LOOKUPS // 128, 128)
    embeds = _sc_gather_build()(table, ids2d)
    e3 = embeds.reshape(B, T, D).transpose(1, 2, 0)
    out_phys = _noise_add(e3)
    return out_phys.transpose(2, 0, 1)


# speedup vs baseline: 1.0795x; 1.0102x over previous
import functools

import jax
import jax.numpy as jnp
from jax import lax
from jax.experimental import pallas as pl
from jax.experimental.pallas import tpu as pltpu
from jax.experimental.pallas import tpu_sc as plsc

B = 4096
T = 200
D = 32
N_LOOKUPS = B * T
N_ELEMS = N_LOOKUPS * D

_NW = 32
_PER_W = N_LOOKUPS // _NW
_IDX_ROWS = _PER_W // 128
_CHUNK = 1280
_NCH = _PER_W // _CHUNK
_GPC = _CHUNK // 128


@functools.lru_cache(maxsize=1)
def _sc_gather_build():
    mesh = plsc.VectorSubcoreMesh(core_axis_name="c", subcore_axis_name="s")

    @functools.partial(
        pl.kernel,
        mesh=mesh,
        compiler_params=pltpu.CompilerParams(use_tc_tiling_on_sc=False),
        out_type=jax.ShapeDtypeStruct((N_LOOKUPS, D), jnp.float32),
        scratch_types=[
            pltpu.VMEM((_IDX_ROWS, 128), jnp.int32),
            pltpu.VMEM((2, _CHUNK, D), jnp.float32),
            pltpu.SemaphoreType.DMA,
            pltpu.SemaphoreType.DMA,
        ],
    )
    def k(table_hbm, ids_hbm, out_hbm, idx_v, rows_v, gsem, wsem):
        wid = lax.axis_index("s") * 2 + lax.axis_index("c")
        pltpu.sync_copy(ids_hbm.at[pl.ds(wid * _IDX_ROWS, _IDX_ROWS)], idx_v)

        def fire(cc, buf):
            handles = []
            for j in range(_GPC):
                handles.append(pltpu.async_copy(
                    table_hbm.at[idx_v.at[cc * _GPC + j]],
                    rows_v.at[buf].at[pl.ds(j * 128, 128)],
                    gsem,
                ))
            return handles

        def drain_and_write(cc, buf, handles):
            for h in handles:
                h.wait()
            return pltpu.async_copy(
                rows_v.at[buf],
                out_hbm.at[pl.ds(wid * _PER_W + cc * _CHUNK, _CHUNK)],
                wsem,
            )

        def pair_body(g, carry):
            ha = fire(2 * g, 0)
            hb = fire(2 * g + 1, 1)
            wa = drain_and_write(2 * g, 0, ha)
            wb = drain_and_write(2 * g + 1, 1, hb)
            wa.wait()
            wb.wait()
            return carry

        lax.fori_loop(0, _NCH // 2, pair_body, 0)

    return k


_TB = 4

_KS0 = 0
_KS1 = 1234
_KS2 = _KS0 ^ _KS1 ^ 0x1BD11BDA
_ROT = ((13, 15, 26, 6), (17, 29, 16, 24))
_MAG = 5.0 / 80.0


def _threefry_bits(x1):
    ks = (jnp.uint32(_KS0), jnp.uint32(_KS1), jnp.uint32(_KS2))
    x0 = jnp.zeros_like(x1) + ks[0]
    x1 = x1 + ks[1]
    for i in range(5):
        for r in _ROT[i % 2]:
            x0 = x0 + x1
            x1 = (x1 << jnp.uint32(r)) | (x1 >> jnp.uint32(32 - r))
            x1 = x0 ^ x1
        x0 = x0 + ks[(i + 1) % 3]
        x1 = x1 + ks[(i + 2) % 3] + jnp.uint32(i + 1)
    return x0 ^ x1


def _noise_add_body(emb_ref, out_ref):
    i = pl.program_id(0)
    t = lax.broadcasted_iota(jnp.int32, (_TB, D, B), 0) + i * _TB
    d = lax.broadcasted_iota(jnp.int32, (_TB, D, B), 1)
    b = lax.broadcasted_iota(jnp.int32, (_TB, D, B), 2)
    f = (b * (T * D) + t * D + d).astype(jnp.uint32)
    bits = _threefry_bits(f)
    fb = (bits >> jnp.uint32(9)) | jnp.uint32(0x3F800000)
    u = lax.bitcast_convert_type(fb, jnp.float32) - jnp.float32(1.0)
    u = jnp.maximum(jnp.float32(-1.0),
                    u * jnp.float32(2.0) - jnp.float32(1.0))
    out_ref[...] = emb_ref[...] + u * jnp.float32(_MAG)


_noise_add = pl.pallas_call(
    _noise_add_body,
    grid=(T // _TB,),
    in_specs=[pl.BlockSpec((_TB, D, B), lambda i: (i, 0, 0))],
    out_specs=pl.BlockSpec((_TB, D, B), lambda i: (i, 0, 0)),
    out_shape=jax.ShapeDtypeStruct((T, D, B), jnp.float32),
)


def kernel(input_ids, table):
    ids2d = input_ids.reshape(N_LOOKUPS // 128, 128)
    embeds = _sc_gather_build()(table, ids2d)
    e3 = embeds.reshape(B, T, D).transpose(1, 2, 0)
    out_phys = _noise_add(e3)
    return out_phys.transpose(2, 0, 1)
